# baseline (speedup 1.0000x reference)
import functools

import jax
import jax.numpy as jnp
import numpy as np
from jax import lax
from jax.experimental import pallas as pl
from jax.experimental.pallas import tpu as pltpu
from jax.experimental.pallas import tpu_sc as plsc

F32 = jnp.float32
I32 = jnp.int32

_N = {"user": 50000, "news": 50000, "source": 10000, "follower": 50000}
_TYPES = ["user", "news", "source", "follower"]
_RELS = [
    ("posts", "user", "news", 400000),
    ("posted_by", "news", "user", 400000),
    ("publishes", "source", "news", 200000),
    ("published_by", "news", "source", 200000),
    ("follows", "follower", "user", 400000),
    ("followed_by", "user", "follower", 400000),
]
_TI = {t: i for i, t in enumerate(_TYPES)}

_BLK = 512


def _rup(x, m):
    return (x + m - 1) // m * m


_P = {t: _rup(_N[t], _BLK) for t in _TYPES}
_HOFF = {}
_off = 0
for _t in _TYPES:
    _HOFF[_t] = _off
    _off += _P[_t]
_H_TOT = _off

_YOFF, _AOFF = [], []
_off_y = _off_a = 0
for _name, _st, _dt, _ne in _RELS:
    _YOFF.append(_off_y)
    _AOFF.append(_off_a)
    _off_y += _P[_st]
    _off_a += _P[_dt]
_Y_TOT = _off_y
_A_TOT = _off_a

_NEP = {400000: 425984, 200000: 229376}
_NSTAGE = {400000: 13, 200000: 7}
_EOFF = []
_off_e = 0
for _name, _st, _dt, _ne in _RELS:
    _EOFF.append(_off_e)
    _off_e += _NEP[_ne]
_E_TOT = _off_e

_ACC_ROWS = 12672
_DUMP = 12544
_PASSES = []
for _r, (_name, _st, _dt, _ne) in enumerate(_RELS):
    _n_dst_p = _P[_dt]
    _npass = _n_dst_p // 12544 if _n_dst_p % 12544 == 0 else 1
    if _n_dst_p == 50176:
        for _k in range(4):
            _PASSES.append((_r, _AOFF[_r] + _k * 12544, 12544, _NSTAGE[_ne]))
    else:
        _PASSES.append((_r, _AOFF[_r], 10240, _NSTAGE[_ne]))
assert len(_PASSES) == 21

_NROUND = 11
_PARAMS = np.zeros((2 * _NROUND, 16), np.int32)
for _i, (_r, _glo, _R, _nst) in enumerate(_PASSES):
    _core, _q = _i % 2, _i // 2
    _row = _core * _NROUND + _q
    _PARAMS[_row, 0] = _EOFF[_r]
    _PARAMS[_row, 1] = _nst
    _PARAMS[_row, 2] = _glo
    _PARAMS[_row, 3] = _glo + _R
    _PARAMS[_row, 4] = _R // 128
    _PARAMS[_row, 5] = 1

_UOFF = []
_USIZE = []
_off_u = 0
for _r, (_name, _st, _dt, _ne) in enumerate(_RELS):
    for _side_t in (_st, _dt):
        if _r == 3 and _side_t == _st:
            _off_u = 262144
        _UOFF.append(_off_u)
        _USIZE.append(_rup(_N[_side_t] + 16, 128))
        _off_u += _USIZE[-1]
_HIST_WORDS = 524288
_IDXH_PER_SC = 2162688
_IDXH_ROWS = 2 * _IDXH_PER_SC // 128

@functools.cache
def _mesh():
    return plsc.VectorSubcoreMesh(core_axis_name="c", subcore_axis_name="s")


def _sel(t, vals):
    expr = jnp.int32(vals[-1])
    for k in range(len(vals) - 2, -1, -1):
        expr = jnp.where(t == k, jnp.int32(vals[k]), expr)
    return expr


def _leaky(v):
    return jnp.where(v > 0, v, 0.01 * v)


_HBLK = [_HOFF[t] // _BLK for t in _TYPES]
_HBND = [_HBLK[1], _HBLK[2], _HBLK[3]]
_H_NBLK = _H_TOT // _BLK
_YBLK = [o // _BLK for o in _YOFF]
_YBND = _YBLK[1:]
_Y_NBLK = _Y_TOT // _BLK
_ABLK = [o // _BLK for o in _AOFF]


def _type_of(i):
    t = jnp.int32(0)
    for b in _HBND:
        t = t + jnp.where(i >= b, 1, 0)
    return t


def _rel_of(i):
    r = jnp.int32(0)
    for b in _YBND:
        r = r + jnp.where(i >= b, 1, 0)
    return r


def _mlp_body(x_ref, w1_ref, b1_ref, w2_ref, b2_ref, o_ref):
    h = jnp.dot(x_ref[...], w1_ref[0], preferred_element_type=F32) + b1_ref[0]
    h = _leaky(h)
    o = jnp.dot(h, w2_ref[0], preferred_element_type=F32) + b2_ref[0]
    o_ref[...] = _leaky(o)


def _run_mlp(x_all, Win1, bin1, Win2, bin2):
    return pl.pallas_call(
        _mlp_body,
        grid=(_H_NBLK,),
        in_specs=[
            pl.BlockSpec((_BLK, 128), lambda i: (i, 0)),
            pl.BlockSpec((1, 128, 128), lambda i: (_type_of(i), 0, 0)),
            pl.BlockSpec((1, 1, 128), lambda i: (_type_of(i), 0, 0)),
            pl.BlockSpec((1, 128, 128), lambda i: (_type_of(i), 0, 0)),
            pl.BlockSpec((1, 1, 128), lambda i: (_type_of(i), 0, 0)),
        ],
        out_specs=pl.BlockSpec((_BLK, 128), lambda i: (i, 0)),
        out_shape=jax.ShapeDtypeStruct((_H_TOT, 128), F32),
    )(x_all, Win1, bin1.reshape(4, 1, 128), Win2, bin2.reshape(4, 1, 128))


def _yt_body(h_ref, w_ref, deg_ref, o_ref):
    deg = deg_ref[...]
    ns = jnp.where(deg > 0, lax.rsqrt(jnp.maximum(deg, 1e-12)), 0.0)
    o_ref[...] = jnp.dot(h_ref[...], w_ref[0], preferred_element_type=F32) * ns


_DH = []
for _r, (_name, _st, _dt, _ne) in enumerate(_RELS):
    _DH.append(_HBLK[_TI[_st]] - _YBLK[_r])


def _run_ytransform(h_all, w_sel, degs):
    return pl.pallas_call(
        _yt_body,
        grid=(_Y_NBLK,),
        in_specs=[
            pl.BlockSpec((_BLK, 128), lambda i: (i + _sel(_rel_of(i), _DH), 0)),
            pl.BlockSpec((1, 128, 128), lambda i: (_rel_of(i), 0, 0)),
            pl.BlockSpec((_BLK, 1), lambda i: (i, 0)),
        ],
        out_specs=pl.BlockSpec((_BLK, 128), lambda i: (i, 0)),
        out_shape=jax.ShapeDtypeStruct((_Y_TOT, 128), F32),
    )(h_all, w_sel, degs)


_PAIR1 = [1, 0, 3, 5]
_PAIR2 = [4, 2, 3, 5]
_D1 = [_ABLK[_PAIR1[t]] - _HBLK[t] for t in range(4)]
_D2 = [_ABLK[_PAIR2[t]] - _HBLK[t] for t in range(4)]


def _nd_of(deg):
    return jnp.where(deg > 0, lax.rsqrt(jnp.maximum(deg, 1e-12)), 0.0)


def _comb_body(a1_ref, a2_ref, d1_ref, d2_ref, bs_ref, o_ref):
    v = (_nd_of(d1_ref[...]) * a1_ref[...]
         + _nd_of(d2_ref[...]) * a2_ref[...]) * 0.5 + bs_ref[0]
    o_ref[...] = _leaky(v)


def _run_combine(agg, degd, bsum):
    return pl.pallas_call(
        _comb_body,
        grid=(_H_NBLK,),
        in_specs=[
            pl.BlockSpec((_BLK, 128), lambda i: (i + _sel(_type_of(i), _D1), 0)),
            pl.BlockSpec((_BLK, 128), lambda i: (i + _sel(_type_of(i), _D2), 0)),
            pl.BlockSpec((_BLK, 1), lambda i: (i + _sel(_type_of(i), _D1), 0)),
            pl.BlockSpec((_BLK, 1), lambda i: (i + _sel(_type_of(i), _D2), 0)),
            pl.BlockSpec((1, 1, 128), lambda i: (_type_of(i), 0, 0)),
        ],
        out_specs=pl.BlockSpec((_BLK, 128), lambda i: (i, 0)),
        out_shape=jax.ShapeDtypeStruct((_H_TOT, 128), F32),
    )(agg, agg, degd, degd, bsum.reshape(4, 1, 128))


def _final_body(a1_ref, a2_ref, d1_ref, d2_ref, bs_ref, wo_ref, bo_ref,
                h1_ref, o_ref):
    v = (_nd_of(d1_ref[...]) * a1_ref[...]
         + _nd_of(d2_ref[...]) * a2_ref[...]) * 0.5 + bs_ref[0]
    h1 = _leaky(v)
    h1_ref[...] = h1
    o_ref[...] = jnp.dot(h1, wo_ref[0], preferred_element_type=F32) + bo_ref[0]


def _run_final(agg, degd, bsum, Wout, bout):
    return pl.pallas_call(
        _final_body,
        grid=(_H_NBLK,),
        in_specs=[
            pl.BlockSpec((_BLK, 128), lambda i: (i + _sel(_type_of(i), _D1), 0)),
            pl.BlockSpec((_BLK, 128), lambda i: (i + _sel(_type_of(i), _D2), 0)),
            pl.BlockSpec((_BLK, 1), lambda i: (i + _sel(_type_of(i), _D1), 0)),
            pl.BlockSpec((_BLK, 1), lambda i: (i + _sel(_type_of(i), _D2), 0)),
            pl.BlockSpec((1, 1, 128), lambda i: (_type_of(i), 0, 0)),
            pl.BlockSpec((1, 128, 64), lambda i: (_type_of(i), 0, 0)),
            pl.BlockSpec((1, 1, 64), lambda i: (_type_of(i), 0, 0)),
        ],
        out_specs=[
            pl.BlockSpec((_BLK, 128), lambda i: (i, 0)),
            pl.BlockSpec((_BLK, 64), lambda i: (i, 0)),
        ],
        out_shape=[
            jax.ShapeDtypeStruct((_H_TOT, 128), F32),
            jax.ShapeDtypeStruct((_H_TOT, 64), F32),
        ],
    )(agg, agg, degd, degd, bsum.reshape(4, 1, 128), Wout,
      bout.reshape(4, 1, 64))


def _hist_body(idx_hbm, zeros_hbm, deg_hbm, hist, zbuf, gidx, ones, sem):
    tid = lax.axis_index("s")
    core = lax.axis_index("c")
    pltpu.sync_copy(zeros_hbm, zbuf)
    for l in range(8):
        ones[pl.ds(l * 16, 16)] = jnp.ones((16,), F32)
    for z in range(2):
        zoff = pl.multiple_of((tid * 2 + z) * 16384, 16384)
        pltpu.sync_copy(zbuf, hist.at[pl.ds(zoff, 16384)])
    plsc.subcore_barrier()

    rowbase = core * 16896 + tid * 1056

    def blk_body(b, _):
        pltpu.sync_copy(idx_hbm.at[pl.ds(rowbase + b * 16, 16)], gidx)
        descs = []
        for j in range(16):
            descs.append(
                pltpu.async_copy(ones, hist.at[gidx.at[j]], sem, add=True))
        for d in descs:
            d.wait()
        return 0

    lax.fori_loop(0, 66, blk_body, 0)
    plsc.subcore_barrier()
    woff = pl.multiple_of(core * 262144 + tid * 16384, 16384)
    pltpu.sync_copy(hist.at[pl.ds(woff, 16384)], deg_hbm.at[pl.ds(woff, 16384)])


@functools.cache
def _hist_kernel_fn():
    return pl.kernel(
        _hist_body,
        out_type=jax.ShapeDtypeStruct((_HIST_WORDS,), F32),
        mesh=_mesh(),
        compiler_params=pltpu.CompilerParams(needs_layout_passes=False),
        scratch_types=[
            pltpu.VMEM_SHARED((_HIST_WORDS,), F32),
            pltpu.VMEM((16384,), F32),
            pltpu.VMEM((16, 128), I32),
            pltpu.VMEM((128,), F32),
            pltpu.SemaphoreType.DMA,
        ],
    )


def _hist_kernel(idxh, z1):
    return _hist_kernel_fn()(idxh, z1)


_IOTA16 = None


def _agg_body(y_hbm, srcg_hbm, dstg_hbm, params_hbm, zeros_hbm, out_hbm,
              acc, sstage, dstage, cpos, gsrc, gdst, rows, pvec,
              sem, sem2, sem3, sem4):
    tid = lax.axis_index("s")
    core = lax.axis_index("c")
    iota = lax.iota(I32, 16)

    def lane(pv, l):
        return jnp.max(jnp.where(iota == l, pv, jnp.int32(-2147483647)))

    for q in range(_NROUND):
        prow = pl.multiple_of((core * _NROUND + q) * 16, 16)
        pltpu.sync_copy(params_hbm.at[pl.ds(prow, 16)], pvec)
        pv = pvec[...]
        eoff = lane(pv, 0)
        nstage = lane(pv, 1)
        glo = lane(pv, 2)
        ghi = lane(pv, 3)
        nunits = lane(pv, 4)
        active = lane(pv, 5)

        @pl.when(active > 0)
        def _pass():
            zsrc = pl.multiple_of(tid * 128, 128)
            for k in range(7):
                u = tid + 16 * k

                @pl.when(u < 99)
                def _z():
                    zo = pl.multiple_of(u * 128, 128)
                    pltpu.sync_copy(zeros_hbm.at[pl.ds(zsrc, 128)],
                                    acc.at[pl.ds(zo, 128)])

            plsc.subcore_barrier()

            chunk = nstage * 2048
            base = eoff + tid * chunk

            for j in range(4):
                sstage[pl.ds(2048 + j * 16, 16)] = tid * 128 + j * 16 + iota
                dstage[pl.ds(2048 + j * 16, 16)] = (
                    glo + _DUMP + j * 16 + iota)

            def stage_body(s, _):
                soff = pl.multiple_of(base + s * 2048, 2048)
                pltpu.sync_copy(srcg_hbm.at[pl.ds(soff, 2048)],
                                sstage.at[pl.ds(0, 2048)])
                pltpu.sync_copy(dstg_hbm.at[pl.ds(soff, 2048)],
                                dstage.at[pl.ds(0, 2048)])

                def filt(i, n):
                    dv = dstage[pl.ds(i * 16, 16)]
                    m = (dv >= glo) & (dv < ghi)
                    plsc.store_compressed(cpos.at[pl.ds(n, 16)],
                                          i * 16 + iota, mask=m)
                    return n + jnp.max(plsc.all_reduce_population_count(m))

                n = lax.fori_loop(0, 128, filt, jnp.int32(0))
                for j in range(4):
                    cpos[pl.ds(n + j * 16, 16)] = 2048 + j * 16 + iota
                nblk = (n + 63) >> 6

                def prep(b, h):
                    for l in range(4):
                        pos = cpos[pl.ds(b * 64 + l * 16, 16)]
                        gsrc[h, pl.ds(l * 16, 16)] = plsc.load_gather(
                            sstage, [pos])
                        gdst[h, pl.ds(l * 16, 16)] = plsc.load_gather(
                            dstage, [pos]) - glo

                def gstart(h, sg):
                    pltpu.async_copy(
                        y_hbm.at[gsrc.at[h]], rows.at[pl.ds(h * 64, 64)], sg)

                def gwait(h, sg):
                    pltpu.make_async_copy(
                        y_hbm.at[gsrc.at[h]], rows.at[pl.ds(h * 64, 64)],
                        sg).wait()

                def sstart(h, ss):
                    pltpu.async_copy(rows.at[pl.ds(h * 64, 64)],
                                     acc.at[gdst.at[h]], ss, add=True)

                def swait(h, ss):
                    pltpu.make_async_copy(rows.at[pl.ds(h * 64, 64)],
                                          acc.at[gdst.at[h]], ss).wait()

                @pl.when(nblk > 0)
                def _prologue():
                    prep(0, 0)
                    gstart(0, sem)

                def pair(m, _):
                    b1 = 2 * m + 1

                    @pl.when(2 * m < nblk)
                    def _even():
                        gwait(0, sem)

                        @pl.when(b1 < nblk)
                        def _p1():
                            @pl.when(m > 0)
                            def _w1():
                                swait(1, sem4)

                            prep(b1, 1)
                            gstart(1, sem2)

                        sstart(0, sem3)

                    @pl.when(b1 < nblk)
                    def _odd():
                        gwait(1, sem2)

                        @pl.when(b1 + 1 < nblk)
                        def _p2():
                            swait(0, sem3)
                            prep(b1 + 1, 0)
                            gstart(0, sem)

                        sstart(1, sem4)

                    return 0

                lax.fori_loop(0, (nblk + 1) >> 1, pair, 0)

                @pl.when(nblk > 0)
                def _d0():
                    swait(0, sem3)

                @pl.when(nblk > 1)
                def _d1():
                    swait(1, sem4)

                return 0

            lax.fori_loop(0, nstage, stage_body, 0)
            plsc.subcore_barrier()

            for k in range(7):
                u = tid + 16 * k

                @pl.when(u < nunits)
                def _wb():
                    ao = pl.multiple_of(u * 128, 128)
                    oo = pl.multiple_of(glo + u * 128, 128)
                    pltpu.sync_copy(
                        acc.at[pl.ds(ao, 128)],
                        out_hbm.at[pl.ds(oo, 128)])

            plsc.subcore_barrier()


@functools.cache
def _agg_kernel_fn():
    return pl.kernel(
        _agg_body,
        out_type=jax.ShapeDtypeStruct((_A_TOT, 128), F32),
        mesh=_mesh(),
        compiler_params=pltpu.CompilerParams(needs_layout_passes=False),
        scratch_types=[
            pltpu.VMEM_SHARED((_ACC_ROWS, 128), F32),
            pltpu.VMEM((2112,), I32),
            pltpu.VMEM((2112,), I32),
            pltpu.VMEM((2176,), I32),
            pltpu.VMEM((2, 64), I32),
            pltpu.VMEM((2, 64), I32),
            pltpu.VMEM((128, 128), F32),
            pltpu.VMEM((16,), I32),
            pltpu.SemaphoreType.DMA,
            pltpu.SemaphoreType.DMA,
            pltpu.SemaphoreType.DMA,
            pltpu.SemaphoreType.DMA,
        ],
    )


def _agg_kernel(y, srcg, dstg, params, z2):
    return _agg_kernel_fn()(y, srcg, dstg, params, z2)


def kernel(x_user, x_news, x_source, x_follower,
           e_posts, e_posted_by, e_publishes, e_published_by, e_follows,
           e_followed_by,
           Win1, bin1, Win2, bin2, Wc1, bc1, Wc2, bc2, Wout, bout):
    xs = {"user": x_user, "news": x_news, "source": x_source,
          "follower": x_follower}
    es = [e_posts, e_posted_by, e_publishes, e_published_by, e_follows,
          e_followed_by]

    x_all = jnp.concatenate(
        [jnp.pad(xs[t], ((0, _P[t] - _N[t]), (0, 0))) for t in _TYPES])

    srcg_parts, dstg_parts, idxh_parts = [], [], []
    for r, (name, st, dt, ne) in enumerate(_RELS):
        src = es[r][0]
        dst = es[r][1]
        npad = _NEP[ne] - ne
        pad_src = (_YOFF[r] + jnp.arange(npad, dtype=I32) % 2048)
        srcg_parts.append(jnp.concatenate([src + _YOFF[r], pad_src]))
        pad_dst = jnp.full((npad,), -1048576, I32)
        dstg_parts.append(jnp.concatenate([dst + _AOFF[r], pad_dst]))
        for side, sidet in ((0, st), (1, dt)):
            u = 2 * r + side
            idx = src if side == 0 else dst
            padh = (_UOFF[u] + _N[sidet]
                    + jnp.arange(npad, dtype=I32) % 16)
            idxh_parts.append(jnp.concatenate([idx + _UOFF[u], padh]))
    srcg = jnp.concatenate(srcg_parts)
    dstg = jnp.concatenate(dstg_parts)
    idxh = jnp.concatenate(idxh_parts).reshape(_IDXH_ROWS, 128)

    params = jnp.asarray(_PARAMS).reshape(-1)
    z1 = jnp.zeros((16384,), F32)
    z2 = jnp.zeros((2048, 128), F32)

    ti_dst = [_TI[dt] for (_, _, dt, _) in _RELS]
    w_sel1 = jnp.stack([Wc1[ti_dst[r], r] for r in range(6)])
    w_sel2 = jnp.stack([Wc2[ti_dst[r], r] for r in range(6)])
    relcnt = [2.0, 2.0, 1.0, 1.0]
    pairs = [(1, 4), (0, 2), (3,), (5,)]
    bsum1 = jnp.stack([sum(bc1[t, r] for r in pairs[t]) / relcnt[t]
                       for t in range(4)])
    bsum2 = jnp.stack([sum(bc2[t, r] for r in pairs[t]) / relcnt[t]
                       for t in range(4)])

    h_all = _run_mlp(x_all, Win1, bin1, Win2, bin2)
    deg_all = _hist_kernel(idxh, z1)

    degs_parts, degd_parts = [], []
    for r, (name, st, dt, ne) in enumerate(_RELS):
        ds_ = deg_all[_UOFF[2 * r]: _UOFF[2 * r] + _N[st]]
        degs_parts.append(jnp.pad(ds_, (0, _P[st] - _N[st])))
        dd_ = deg_all[_UOFF[2 * r + 1]: _UOFF[2 * r + 1] + _N[dt]]
        degd_parts.append(jnp.pad(dd_, (0, _P[dt] - _N[dt])))
    degs = jnp.concatenate(degs_parts).reshape(_Y_TOT, 1)
    degd = jnp.concatenate(degd_parts).reshape(_A_TOT, 1)

    y1 = _run_ytransform(h_all, w_sel1, degs)
    agg1 = _agg_kernel(y1, srcg, dstg, params, z2)
    h2 = _run_combine(agg1, degd, bsum1)
    y2 = _run_ytransform(h2, w_sel2, degs)
    agg2 = _agg_kernel(y2, srcg, dstg, params, z2)
    h1f, outf = _run_final(agg2, degd, bsum2, Wout, bout)

    res_out, res_h1 = [], []
    for t in _TYPES:
        res_out.append(outf[_HOFF[t]: _HOFF[t] + _N[t]])
        res_h1.append(h1f[_HOFF[t]: _HOFF[t] + _N[t]])
    return (*res_out, *res_h1)


# speedup vs baseline: 1.0266x; 1.0266x over previous
import functools

import jax
import jax.numpy as jnp
import numpy as np
from jax import lax
from jax.experimental import pallas as pl
from jax.experimental.pallas import tpu as pltpu
from jax.experimental.pallas import tpu_sc as plsc

F32 = jnp.float32
I32 = jnp.int32

_N = {"user": 50000, "news": 50000, "source": 10000, "follower": 50000}
_TYPES = ["user", "news", "source", "follower"]
_RELS = [
    ("posts", "user", "news", 400000),
    ("posted_by", "news", "user", 400000),
    ("publishes", "source", "news", 200000),
    ("published_by", "news", "source", 200000),
    ("follows", "follower", "user", 400000),
    ("followed_by", "user", "follower", 400000),
]
_TI = {t: i for i, t in enumerate(_TYPES)}

_BLK = 512


def _rup(x, m):
    return (x + m - 1) // m * m


_P = {t: _rup(_N[t], _BLK) for t in _TYPES}
_HOFF = {}
_off = 0
for _t in _TYPES:
    _HOFF[_t] = _off
    _off += _P[_t]
_H_TOT = _off

_YOFF, _AOFF = [], []
_off_y = _off_a = 0
for _name, _st, _dt, _ne in _RELS:
    _YOFF.append(_off_y)
    _AOFF.append(_off_a)
    _off_y += _P[_st]
    _off_a += _P[_dt]
_Y_TOT = _off_y
_A_TOT = _off_a

_NEP = {400000: 425984, 200000: 229376}
_NSTAGE = {400000: 13, 200000: 7}
_EOFF = []
_off_e = 0
for _name, _st, _dt, _ne in _RELS:
    _EOFF.append(_off_e)
    _off_e += _NEP[_ne]
_E_TOT = _off_e

_ACC_ROWS = 12672
_DUMP = 12544
_PASSES = []
for _r, (_name, _st, _dt, _ne) in enumerate(_RELS):
    _n_dst_p = _P[_dt]
    _npass = _n_dst_p // 12544 if _n_dst_p % 12544 == 0 else 1
    if _n_dst_p == 50176:
        for _k in range(4):
            _PASSES.append((_r, _AOFF[_r] + _k * 12544, 12544, _NSTAGE[_ne]))
    else:
        _PASSES.append((_r, _AOFF[_r], 10240, _NSTAGE[_ne]))
assert len(_PASSES) == 21

_NROUND = 11
_PARAMS = np.zeros((2 * _NROUND, 16), np.int32)
for _i, (_r, _glo, _R, _nst) in enumerate(_PASSES):
    _core, _q = _i % 2, _i // 2
    _row = _core * _NROUND + _q
    _PARAMS[_row, 0] = _EOFF[_r]
    _PARAMS[_row, 1] = _nst
    _PARAMS[_row, 2] = _glo
    _PARAMS[_row, 3] = _glo + _R
    _PARAMS[_row, 4] = _R // 128
    _PARAMS[_row, 5] = 1

_UOFF = []
_USIZE = []
_off_u = 0
for _r, (_name, _st, _dt, _ne) in enumerate(_RELS):
    for _side_t in (_st, _dt):
        if _r == 3 and _side_t == _st:
            _off_u = 262144
        _UOFF.append(_off_u)
        _USIZE.append(_rup(_N[_side_t] + 16, 128))
        _off_u += _USIZE[-1]
_HIST_WORDS = 524288
_IDXH_PER_SC = 2162688
_IDXH_ROWS = 2 * _IDXH_PER_SC // 128

@functools.cache
def _mesh():
    return plsc.VectorSubcoreMesh(core_axis_name="c", subcore_axis_name="s")


def _sel(t, vals):
    expr = jnp.int32(vals[-1])
    for k in range(len(vals) - 2, -1, -1):
        expr = jnp.where(t == k, jnp.int32(vals[k]), expr)
    return expr


def _leaky(v):
    return jnp.where(v > 0, v, 0.01 * v)


_HBLK = [_HOFF[t] // _BLK for t in _TYPES]
_HBND = [_HBLK[1], _HBLK[2], _HBLK[3]]
_H_NBLK = _H_TOT // _BLK
_YBLK = [o // _BLK for o in _YOFF]
_YBND = _YBLK[1:]
_Y_NBLK = _Y_TOT // _BLK
_ABLK = [o // _BLK for o in _AOFF]


def _type_of(i):
    t = jnp.int32(0)
    for b in _HBND:
        t = t + jnp.where(i >= b, 1, 0)
    return t


def _rel_of(i):
    r = jnp.int32(0)
    for b in _YBND:
        r = r + jnp.where(i >= b, 1, 0)
    return r


def _mlp_body(x_ref, w1_ref, b1_ref, w2_ref, b2_ref, o_ref):
    h = jnp.dot(x_ref[...], w1_ref[0], preferred_element_type=F32) + b1_ref[0]
    h = _leaky(h)
    o = jnp.dot(h, w2_ref[0], preferred_element_type=F32) + b2_ref[0]
    o_ref[...] = _leaky(o)


def _run_mlp(x_all, Win1, bin1, Win2, bin2):
    return pl.pallas_call(
        _mlp_body,
        grid=(_H_NBLK,),
        in_specs=[
            pl.BlockSpec((_BLK, 128), lambda i: (i, 0)),
            pl.BlockSpec((1, 128, 128), lambda i: (_type_of(i), 0, 0)),
            pl.BlockSpec((1, 1, 128), lambda i: (_type_of(i), 0, 0)),
            pl.BlockSpec((1, 128, 128), lambda i: (_type_of(i), 0, 0)),
            pl.BlockSpec((1, 1, 128), lambda i: (_type_of(i), 0, 0)),
        ],
        out_specs=pl.BlockSpec((_BLK, 128), lambda i: (i, 0)),
        out_shape=jax.ShapeDtypeStruct((_H_TOT, 128), F32),
    )(x_all, Win1, bin1.reshape(4, 1, 128), Win2, bin2.reshape(4, 1, 128))


def _yt_body(h_ref, w_ref, deg_ref, o_ref):
    deg = deg_ref[...]
    ns = jnp.where(deg > 0, lax.rsqrt(jnp.maximum(deg, 1e-12)), 0.0)
    o_ref[...] = jnp.dot(h_ref[...], w_ref[0], preferred_element_type=F32) * ns


_DH = []
for _r, (_name, _st, _dt, _ne) in enumerate(_RELS):
    _DH.append(_HBLK[_TI[_st]] - _YBLK[_r])


def _run_ytransform(h_all, w_sel, degs):
    return pl.pallas_call(
        _yt_body,
        grid=(_Y_NBLK,),
        in_specs=[
            pl.BlockSpec((_BLK, 128), lambda i: (i + _sel(_rel_of(i), _DH), 0)),
            pl.BlockSpec((1, 128, 128), lambda i: (_rel_of(i), 0, 0)),
            pl.BlockSpec((_BLK, 1), lambda i: (i, 0)),
        ],
        out_specs=pl.BlockSpec((_BLK, 128), lambda i: (i, 0)),
        out_shape=jax.ShapeDtypeStruct((_Y_TOT, 128), F32),
    )(h_all, w_sel, degs)


_PAIR1 = [1, 0, 3, 5]
_PAIR2 = [4, 2, 3, 5]
_D1 = [_ABLK[_PAIR1[t]] - _HBLK[t] for t in range(4)]
_D2 = [_ABLK[_PAIR2[t]] - _HBLK[t] for t in range(4)]
_TIST = [_TI[st] for (_name, st, _dt, _ne) in _RELS]
_DA1 = [_DH[r] + _D1[_TIST[r]] for r in range(6)]
_DA2 = [_DH[r] + _D2[_TIST[r]] for r in range(6)]


def _cyt_body(a1_ref, a2_ref, d1_ref, d2_ref, bs_ref, w_ref, degs_ref,
              o_ref):
    h = _leaky((_nd_of(d1_ref[...]) * a1_ref[...]
                + _nd_of(d2_ref[...]) * a2_ref[...]) * 0.5 + bs_ref[0])
    deg = degs_ref[...]
    ns = jnp.where(deg > 0, lax.rsqrt(jnp.maximum(deg, 1e-12)), 0.0)
    o_ref[...] = jnp.dot(h, w_ref[0], preferred_element_type=F32) * ns


def _run_combine_ytransform(agg, degd, bsum, w_sel, degs):
    return pl.pallas_call(
        _cyt_body,
        grid=(_Y_NBLK,),
        in_specs=[
            pl.BlockSpec((_BLK, 128), lambda i: (i + _sel(_rel_of(i), _DA1), 0)),
            pl.BlockSpec((_BLK, 128), lambda i: (i + _sel(_rel_of(i), _DA2), 0)),
            pl.BlockSpec((_BLK, 1), lambda i: (i + _sel(_rel_of(i), _DA1), 0)),
            pl.BlockSpec((_BLK, 1), lambda i: (i + _sel(_rel_of(i), _DA2), 0)),
            pl.BlockSpec((1, 1, 128), lambda i: (_sel(_rel_of(i), _TIST), 0, 0)),
            pl.BlockSpec((1, 128, 128), lambda i: (_rel_of(i), 0, 0)),
            pl.BlockSpec((_BLK, 1), lambda i: (i, 0)),
        ],
        out_specs=pl.BlockSpec((_BLK, 128), lambda i: (i, 0)),
        out_shape=jax.ShapeDtypeStruct((_Y_TOT, 128), F32),
    )(agg, agg, degd, degd, bsum.reshape(4, 1, 128), w_sel, degs)


def _nd_of(deg):
    return jnp.where(deg > 0, lax.rsqrt(jnp.maximum(deg, 1e-12)), 0.0)


def _comb_body(a1_ref, a2_ref, d1_ref, d2_ref, bs_ref, o_ref):
    v = (_nd_of(d1_ref[...]) * a1_ref[...]
         + _nd_of(d2_ref[...]) * a2_ref[...]) * 0.5 + bs_ref[0]
    o_ref[...] = _leaky(v)


def _run_combine(agg, degd, bsum):
    return pl.pallas_call(
        _comb_body,
        grid=(_H_NBLK,),
        in_specs=[
            pl.BlockSpec((_BLK, 128), lambda i: (i + _sel(_type_of(i), _D1), 0)),
            pl.BlockSpec((_BLK, 128), lambda i: (i + _sel(_type_of(i), _D2), 0)),
            pl.BlockSpec((_BLK, 1), lambda i: (i + _sel(_type_of(i), _D1), 0)),
            pl.BlockSpec((_BLK, 1), lambda i: (i + _sel(_type_of(i), _D2), 0)),
            pl.BlockSpec((1, 1, 128), lambda i: (_type_of(i), 0, 0)),
        ],
        out_specs=pl.BlockSpec((_BLK, 128), lambda i: (i, 0)),
        out_shape=jax.ShapeDtypeStruct((_H_TOT, 128), F32),
    )(agg, agg, degd, degd, bsum.reshape(4, 1, 128))


def _final_body(a1_ref, a2_ref, d1_ref, d2_ref, bs_ref, wo_ref, bo_ref,
                h1_ref, o_ref):
    v = (_nd_of(d1_ref[...]) * a1_ref[...]
         + _nd_of(d2_ref[...]) * a2_ref[...]) * 0.5 + bs_ref[0]
    h1 = _leaky(v)
    h1_ref[...] = h1
    o_ref[...] = jnp.dot(h1, wo_ref[0], preferred_element_type=F32) + bo_ref[0]


def _run_final(agg, degd, bsum, Wout, bout):
    return pl.pallas_call(
        _final_body,
        grid=(_H_NBLK,),
        in_specs=[
            pl.BlockSpec((_BLK, 128), lambda i: (i + _sel(_type_of(i), _D1), 0)),
            pl.BlockSpec((_BLK, 128), lambda i: (i + _sel(_type_of(i), _D2), 0)),
            pl.BlockSpec((_BLK, 1), lambda i: (i + _sel(_type_of(i), _D1), 0)),
            pl.BlockSpec((_BLK, 1), lambda i: (i + _sel(_type_of(i), _D2), 0)),
            pl.BlockSpec((1, 1, 128), lambda i: (_type_of(i), 0, 0)),
            pl.BlockSpec((1, 128, 64), lambda i: (_type_of(i), 0, 0)),
            pl.BlockSpec((1, 1, 64), lambda i: (_type_of(i), 0, 0)),
        ],
        out_specs=[
            pl.BlockSpec((_BLK, 128), lambda i: (i, 0)),
            pl.BlockSpec((_BLK, 64), lambda i: (i, 0)),
        ],
        out_shape=[
            jax.ShapeDtypeStruct((_H_TOT, 128), F32),
            jax.ShapeDtypeStruct((_H_TOT, 64), F32),
        ],
    )(agg, agg, degd, degd, bsum.reshape(4, 1, 128), Wout,
      bout.reshape(4, 1, 64))


def _hist_body(idx_hbm, zeros_hbm, deg_hbm, hist, zbuf, gidx, ones, sem):
    tid = lax.axis_index("s")
    core = lax.axis_index("c")
    pltpu.sync_copy(zeros_hbm, zbuf)
    for l in range(8):
        ones[pl.ds(l * 16, 16)] = jnp.ones((16,), F32)
    for z in range(2):
        zoff = pl.multiple_of((tid * 2 + z) * 16384, 16384)
        pltpu.sync_copy(zbuf, hist.at[pl.ds(zoff, 16384)])
    plsc.subcore_barrier()

    rowbase = core * 16896 + tid * 1056

    def blk_body(b, _):
        pltpu.sync_copy(idx_hbm.at[pl.ds(rowbase + b * 16, 16)], gidx)
        descs = []
        for j in range(16):
            descs.append(
                pltpu.async_copy(ones, hist.at[gidx.at[j]], sem, add=True))
        for d in descs:
            d.wait()
        return 0

    lax.fori_loop(0, 66, blk_body, 0)
    plsc.subcore_barrier()
    woff = pl.multiple_of(core * 262144 + tid * 16384, 16384)
    pltpu.sync_copy(hist.at[pl.ds(woff, 16384)], deg_hbm.at[pl.ds(woff, 16384)])


@functools.cache
def _hist_kernel_fn():
    return pl.kernel(
        _hist_body,
        out_type=jax.ShapeDtypeStruct((_HIST_WORDS,), F32),
        mesh=_mesh(),
        compiler_params=pltpu.CompilerParams(needs_layout_passes=False),
        scratch_types=[
            pltpu.VMEM_SHARED((_HIST_WORDS,), F32),
            pltpu.VMEM((16384,), F32),
            pltpu.VMEM((16, 128), I32),
            pltpu.VMEM((128,), F32),
            pltpu.SemaphoreType.DMA,
        ],
    )


def _hist_kernel(idxh, z1):
    return _hist_kernel_fn()(idxh, z1)


_IOTA16 = None


def _agg_body(y_hbm, srcg_hbm, dstg_hbm, params_hbm, zeros_hbm, out_hbm,
              acc, sstage, dstage, cpos, gsrc, gdst, rows, pvec,
              sem, sem2, sem3, sem4):
    tid = lax.axis_index("s")
    core = lax.axis_index("c")
    iota = lax.iota(I32, 16)

    def lane(pv, l):
        return jnp.max(jnp.where(iota == l, pv, jnp.int32(-2147483647)))

    for q in range(_NROUND):
        prow = pl.multiple_of((core * _NROUND + q) * 16, 16)
        pltpu.sync_copy(params_hbm.at[pl.ds(prow, 16)], pvec)
        pv = pvec[...]
        eoff = lane(pv, 0)
        nstage = lane(pv, 1)
        glo = lane(pv, 2)
        ghi = lane(pv, 3)
        nunits = lane(pv, 4)
        active = lane(pv, 5)

        @pl.when(active > 0)
        def _pass():
            zsrc = pl.multiple_of(tid * 128, 128)
            for k in range(7):
                u = tid + 16 * k

                @pl.when(u < 99)
                def _z():
                    zo = pl.multiple_of(u * 128, 128)
                    pltpu.sync_copy(zeros_hbm.at[pl.ds(zsrc, 128)],
                                    acc.at[pl.ds(zo, 128)])

            plsc.subcore_barrier()

            chunk = nstage * 2048
            base = eoff + tid * chunk

            for j in range(4):
                sstage[pl.ds(2048 + j * 16, 16)] = tid * 128 + j * 16 + iota
                dstage[pl.ds(2048 + j * 16, 16)] = (
                    glo + _DUMP + j * 16 + iota)

            def stage_body(s, _):
                soff = pl.multiple_of(base + s * 2048, 2048)
                pltpu.sync_copy(srcg_hbm.at[pl.ds(soff, 2048)],
                                sstage.at[pl.ds(0, 2048)])
                pltpu.sync_copy(dstg_hbm.at[pl.ds(soff, 2048)],
                                dstage.at[pl.ds(0, 2048)])

                def filt(i, n):
                    dv = dstage[pl.ds(i * 16, 16)]
                    m = (dv >= glo) & (dv < ghi)
                    plsc.store_compressed(cpos.at[pl.ds(n, 16)],
                                          i * 16 + iota, mask=m)
                    return n + jnp.max(plsc.all_reduce_population_count(m))

                n = lax.fori_loop(0, 128, filt, jnp.int32(0))
                for j in range(4):
                    cpos[pl.ds(n + j * 16, 16)] = 2048 + j * 16 + iota
                nblk = (n + 63) >> 6

                def prep(b, h):
                    for l in range(4):
                        pos = cpos[pl.ds(b * 64 + l * 16, 16)]
                        gsrc[h, pl.ds(l * 16, 16)] = plsc.load_gather(
                            sstage, [pos])
                        gdst[h, pl.ds(l * 16, 16)] = plsc.load_gather(
                            dstage, [pos]) - glo

                def gstart(h, sg):
                    pltpu.async_copy(
                        y_hbm.at[gsrc.at[h]], rows.at[pl.ds(h * 64, 64)], sg)

                def gwait(h, sg):
                    pltpu.make_async_copy(
                        y_hbm.at[gsrc.at[h]], rows.at[pl.ds(h * 64, 64)],
                        sg).wait()

                def sstart(h, ss):
                    pltpu.async_copy(rows.at[pl.ds(h * 64, 64)],
                                     acc.at[gdst.at[h]], ss, add=True)

                def swait(h, ss):
                    pltpu.make_async_copy(rows.at[pl.ds(h * 64, 64)],
                                          acc.at[gdst.at[h]], ss).wait()

                @pl.when(nblk > 0)
                def _prologue():
                    prep(0, 0)
                    gstart(0, sem)

                def pair(m, _):
                    b1 = 2 * m + 1

                    @pl.when(2 * m < nblk)
                    def _even():
                        gwait(0, sem)

                        @pl.when(b1 < nblk)
                        def _p1():
                            @pl.when(m > 0)
                            def _w1():
                                swait(1, sem4)

                            prep(b1, 1)
                            gstart(1, sem2)

                        sstart(0, sem3)

                    @pl.when(b1 < nblk)
                    def _odd():
                        gwait(1, sem2)

                        @pl.when(b1 + 1 < nblk)
                        def _p2():
                            swait(0, sem3)
                            prep(b1 + 1, 0)
                            gstart(0, sem)

                        sstart(1, sem4)

                    return 0

                lax.fori_loop(0, (nblk + 1) >> 1, pair, 0)

                @pl.when(nblk > 0)
                def _d0():
                    swait(0, sem3)

                @pl.when(nblk > 1)
                def _d1():
                    swait(1, sem4)

                return 0

            lax.fori_loop(0, nstage, stage_body, 0)
            plsc.subcore_barrier()

            for k in range(7):
                u = tid + 16 * k

                @pl.when(u < nunits)
                def _wb():
                    ao = pl.multiple_of(u * 128, 128)
                    oo = pl.multiple_of(glo + u * 128, 128)
                    pltpu.sync_copy(
                        acc.at[pl.ds(ao, 128)],
                        out_hbm.at[pl.ds(oo, 128)])

            plsc.subcore_barrier()


@functools.cache
def _agg_kernel_fn():
    return pl.kernel(
        _agg_body,
        out_type=jax.ShapeDtypeStruct((_A_TOT, 128), F32),
        mesh=_mesh(),
        compiler_params=pltpu.CompilerParams(needs_layout_passes=False),
        scratch_types=[
            pltpu.VMEM_SHARED((_ACC_ROWS, 128), F32),
            pltpu.VMEM((2112,), I32),
            pltpu.VMEM((2112,), I32),
            pltpu.VMEM((2176,), I32),
            pltpu.VMEM((2, 64), I32),
            pltpu.VMEM((2, 64), I32),
            pltpu.VMEM((128, 128), F32),
            pltpu.VMEM((16,), I32),
            pltpu.SemaphoreType.DMA,
            pltpu.SemaphoreType.DMA,
            pltpu.SemaphoreType.DMA,
            pltpu.SemaphoreType.DMA,
        ],
    )


def _agg_kernel(y, srcg, dstg, params, z2):
    return _agg_kernel_fn()(y, srcg, dstg, params, z2)


def kernel(x_user, x_news, x_source, x_follower,
           e_posts, e_posted_by, e_publishes, e_published_by, e_follows,
           e_followed_by,
           Win1, bin1, Win2, bin2, Wc1, bc1, Wc2, bc2, Wout, bout):
    xs = {"user": x_user, "news": x_news, "source": x_source,
          "follower": x_follower}
    es = [e_posts, e_posted_by, e_publishes, e_published_by, e_follows,
          e_followed_by]

    x_all = jnp.concatenate(
        [jnp.pad(xs[t], ((0, _P[t] - _N[t]), (0, 0))) for t in _TYPES])

    srcg_parts, dstg_parts, idxh_parts = [], [], []
    for r, (name, st, dt, ne) in enumerate(_RELS):
        src = es[r][0]
        dst = es[r][1]
        npad = _NEP[ne] - ne
        pad_src = (_YOFF[r] + jnp.arange(npad, dtype=I32) % 2048)
        srcg_parts.append(jnp.concatenate([src + _YOFF[r], pad_src]))
        pad_dst = jnp.full((npad,), -1048576, I32)
        dstg_parts.append(jnp.concatenate([dst + _AOFF[r], pad_dst]))
        for side, sidet in ((0, st), (1, dt)):
            u = 2 * r + side
            idx = src if side == 0 else dst
            padh = (_UOFF[u] + _N[sidet]
                    + jnp.arange(npad, dtype=I32) % 16)
            idxh_parts.append(jnp.concatenate([idx + _UOFF[u], padh]))
    srcg = jnp.concatenate(srcg_parts)
    dstg = jnp.concatenate(dstg_parts)
    idxh = jnp.concatenate(idxh_parts).reshape(_IDXH_ROWS, 128)

    params = jnp.asarray(_PARAMS).reshape(-1)
    z1 = jnp.zeros((16384,), F32)
    z2 = jnp.zeros((2048, 128), F32)

    ti_dst = [_TI[dt] for (_, _, dt, _) in _RELS]
    w_sel1 = jnp.stack([Wc1[ti_dst[r], r] for r in range(6)])
    w_sel2 = jnp.stack([Wc2[ti_dst[r], r] for r in range(6)])
    relcnt = [2.0, 2.0, 1.0, 1.0]
    pairs = [(1, 4), (0, 2), (3,), (5,)]
    bsum1 = jnp.stack([sum(bc1[t, r] for r in pairs[t]) / relcnt[t]
                       for t in range(4)])
    bsum2 = jnp.stack([sum(bc2[t, r] for r in pairs[t]) / relcnt[t]
                       for t in range(4)])

    h_all = _run_mlp(x_all, Win1, bin1, Win2, bin2)
    deg_all = _hist_kernel(idxh, z1)

    degs_parts, degd_parts = [], []
    for r, (name, st, dt, ne) in enumerate(_RELS):
        ds_ = deg_all[_UOFF[2 * r]: _UOFF[2 * r] + _N[st]]
        degs_parts.append(jnp.pad(ds_, (0, _P[st] - _N[st])))
        dd_ = deg_all[_UOFF[2 * r + 1]: _UOFF[2 * r + 1] + _N[dt]]
        degd_parts.append(jnp.pad(dd_, (0, _P[dt] - _N[dt])))
    degs = jnp.concatenate(degs_parts).reshape(_Y_TOT, 1)
    degd = jnp.concatenate(degd_parts).reshape(_A_TOT, 1)

    y1 = _run_ytransform(h_all, w_sel1, degs)
    agg1 = _agg_kernel(y1, srcg, dstg, params, z2)
    y2 = _run_combine_ytransform(agg1, degd, bsum1, w_sel2, degs)
    agg2 = _agg_kernel(y2, srcg, dstg, params, z2)
    h1f, outf = _run_final(agg2, degd, bsum2, Wout, bout)

    res_out, res_h1 = [], []
    for t in _TYPES:
        res_out.append(outf[_HOFF[t]: _HOFF[t] + _N[t]])
        res_h1.append(h1f[_HOFF[t]: _HOFF[t] + _N[t]])
    return (*res_out, *res_h1)
